# baseline (speedup 1.0000x reference)
import functools

import jax
import jax.numpy as jnp
from jax import lax
from jax.experimental import pallas as pl
from jax.experimental.pallas import tpu as pltpu
from jax.experimental.pallas import tpu_sc as plsc

VOCAB = 1000
HIDDEN = 128
N_TOKENS = 4096 * 20

_INFO = plsc.get_sparse_core_info()
NC, NS = _INFO.num_cores, _INFO.num_subcores
NW = NC * NS
B_PER_W = N_TOKENS // NW
CHUNK = 40
N_CHUNKS = B_PER_W // CHUNK


def _proj_table_kernel(emb_ref, w_ref, b_ref, out_ref):
    out_ref[...] = lax.dot_general(
        emb_ref[...], w_ref[...],
        (((1,), (1,)), ((), ())),
        preferred_element_type=jnp.float32,
        precision=lax.Precision.HIGHEST,
    ) + b_ref[...]


def _make_proj_table(emb_table, W, b):
    return pl.pallas_call(
        _proj_table_kernel,
        out_shape=jax.ShapeDtypeStruct((VOCAB, VOCAB), jnp.float32),
    )(emb_table, W, b.reshape(1, VOCAB))


_sc_mesh = plsc.VectorSubcoreMesh(core_axis_name="c", subcore_axis_name="s")


@functools.partial(
    pl.kernel,
    out_type=jax.ShapeDtypeStruct((N_TOKENS, VOCAB), jnp.float32),
    mesh=_sc_mesh,
    scratch_types=[
        pltpu.VMEM((B_PER_W,), jnp.int32),
        pltpu.VMEM((CHUNK, VOCAB), jnp.float32),
        pltpu.SemaphoreType.DMA,
    ],
    compiler_params=pltpu.CompilerParams(use_tc_tiling_on_sc=False),
)
def _sc_gather(ids_hbm, p_hbm, out_hbm, idx_v, rows_v, sem):
    wid = lax.axis_index("s") * NC + lax.axis_index("c")
    base = wid * B_PER_W
    pltpu.sync_copy(ids_hbm.at[pl.ds(base, B_PER_W)], idx_v)

    def chunk_body(i, carry):
        off = i * CHUNK
        pltpu.async_copy(
            p_hbm.at[idx_v.at[pl.ds(off, CHUNK)]], rows_v, sem
        ).wait()
        pltpu.sync_copy(rows_v, out_hbm.at[pl.ds(base + off, CHUNK)])
        return carry

    lax.fori_loop(0, N_CHUNKS, chunk_body, 0)


def kernel(input_ids, positions, emb_table, W, b):
    del positions
    P = _make_proj_table(emb_table, W, b)
    ids = input_ids.reshape(-1).astype(jnp.int32)
    out = _sc_gather(ids, P)
    return out.reshape(input_ids.shape[0], input_ids.shape[1], VOCAB)


# speedup vs baseline: 1.0422x; 1.0422x over previous
import functools

import jax
import jax.numpy as jnp
from jax import lax
from jax.experimental import pallas as pl
from jax.experimental.pallas import tpu as pltpu
from jax.experimental.pallas import tpu_sc as plsc

VOCAB = 1000
HIDDEN = 128
N_TOKENS = 4096 * 20

_INFO = plsc.get_sparse_core_info()
NC, NS = _INFO.num_cores, _INFO.num_subcores
NW = NC * NS
B_PER_W = N_TOKENS // NW
CHUNK = 40
N_CHUNKS = B_PER_W // CHUNK


def _proj_table_kernel(emb_ref, w_ref, b_ref, out_ref):
    out_ref[...] = lax.dot_general(
        emb_ref[...], w_ref[...],
        (((1,), (1,)), ((), ())),
        preferred_element_type=jnp.float32,
        precision=lax.Precision.HIGHEST,
    ) + b_ref[...]


def _make_proj_table(emb_table, W, b):
    return pl.pallas_call(
        _proj_table_kernel,
        out_shape=jax.ShapeDtypeStruct((VOCAB, VOCAB), jnp.float32),
    )(emb_table, W, b.reshape(1, VOCAB))


_sc_mesh = plsc.VectorSubcoreMesh(core_axis_name="c", subcore_axis_name="s")


@functools.partial(
    pl.kernel,
    out_type=jax.ShapeDtypeStruct((N_TOKENS, VOCAB), jnp.float32),
    mesh=_sc_mesh,
    scratch_types=[
        pltpu.VMEM((B_PER_W,), jnp.int32),
        pltpu.VMEM((CHUNK, VOCAB), jnp.float32),
        pltpu.VMEM((CHUNK, VOCAB), jnp.float32),
        pltpu.SemaphoreType.DMA,
        pltpu.SemaphoreType.DMA,
        pltpu.SemaphoreType.DMA,
        pltpu.SemaphoreType.DMA,
    ],
    compiler_params=pltpu.CompilerParams(use_tc_tiling_on_sc=False),
)
def _sc_gather(ids_hbm, p_hbm, out_hbm, idx_v, rows0, rows1,
               gsem0, gsem1, wsem0, wsem1):
    wid = lax.axis_index("s") * NC + lax.axis_index("c")
    base = wid * B_PER_W
    pltpu.sync_copy(ids_hbm.at[pl.ds(base, B_PER_W)], idx_v)
    rows = (rows0, rows1)
    gsem = (gsem0, gsem1)
    wsem = (wsem0, wsem1)

    def outer_body(j, carry):
        for slot in range(2):
            i = 2 * j + slot
            off = i * CHUNK

            @pl.when(j > 0)
            def _wait_prev_writeback():
                pltpu.make_async_copy(
                    rows[slot], out_hbm.at[pl.ds(base, CHUNK)], wsem[slot]
                ).wait()

            pltpu.async_copy(
                p_hbm.at[idx_v.at[pl.ds(off, CHUNK)]], rows[slot], gsem[slot]
            ).wait()
            pltpu.async_copy(
                rows[slot], out_hbm.at[pl.ds(base + off, CHUNK)], wsem[slot]
            )
        return carry

    lax.fori_loop(0, N_CHUNKS // 2, outer_body, 0)
    for slot in range(2):
        pltpu.make_async_copy(
            rows[slot], out_hbm.at[pl.ds(base, CHUNK)], wsem[slot]
        ).wait()


def kernel(input_ids, positions, emb_table, W, b):
    del positions
    P = _make_proj_table(emb_table, W, b)
    ids = input_ids.reshape(-1).astype(jnp.int32)
    out = _sc_gather(ids, P)
    return out.reshape(input_ids.shape[0], input_ids.shape[1], VOCAB)


# speedup vs baseline: 1.0630x; 1.0200x over previous
import functools

import jax
import jax.numpy as jnp
from jax import lax
from jax.experimental import pallas as pl
from jax.experimental.pallas import tpu as pltpu
from jax.experimental.pallas import tpu_sc as plsc

VOCAB = 1000
HIDDEN = 128
N_TOKENS = 4096 * 20

_INFO = plsc.get_sparse_core_info()
NC, NS = _INFO.num_cores, _INFO.num_subcores
NW = NC * NS
B_PER_W = N_TOKENS // NW
CHUNK = 256
N_CHUNKS = B_PER_W // CHUNK

_sc_mesh = plsc.VectorSubcoreMesh(core_axis_name="c", subcore_axis_name="s")


@functools.partial(
    pl.kernel,
    out_type=jax.ShapeDtypeStruct((N_TOKENS, HIDDEN), jnp.float32),
    mesh=_sc_mesh,
    scratch_types=[
        pltpu.VMEM((B_PER_W,), jnp.int32),
        pltpu.VMEM((CHUNK, HIDDEN), jnp.float32),
        pltpu.VMEM((CHUNK, HIDDEN), jnp.float32),
        pltpu.SemaphoreType.DMA,
        pltpu.SemaphoreType.DMA,
        pltpu.SemaphoreType.DMA,
        pltpu.SemaphoreType.DMA,
    ],
)
def _sc_gather(ids_hbm, emb_hbm, out_hbm, idx_v, rows0, rows1,
               gsem0, gsem1, wsem0, wsem1):
    wid = lax.axis_index("s") * NC + lax.axis_index("c")
    base = wid * B_PER_W
    pltpu.sync_copy(ids_hbm.at[pl.ds(base, B_PER_W)], idx_v)
    rows = (rows0, rows1)
    gsem = (gsem0, gsem1)
    wsem = (wsem0, wsem1)

    def outer_body(j, carry):
        for slot in range(2):
            i = 2 * j + slot
            off = i * CHUNK

            @pl.when(j > 0)
            def _wait_prev_writeback():
                pltpu.make_async_copy(
                    rows[slot], out_hbm.at[pl.ds(base, CHUNK)], wsem[slot]
                ).wait()

            pltpu.async_copy(
                emb_hbm.at[idx_v.at[pl.ds(off, CHUNK)]], rows[slot], gsem[slot]
            ).wait()
            pltpu.async_copy(
                rows[slot], out_hbm.at[pl.ds(base + off, CHUNK)], wsem[slot]
            )
        return carry

    lax.fori_loop(0, N_CHUNKS // 2, outer_body, 0)
    for slot in range(2):
        pltpu.make_async_copy(
            rows[slot], out_hbm.at[pl.ds(base, CHUNK)], wsem[slot]
        ).wait()


BT = 512
N_BLOCKS = N_TOKENS // BT


def _proj_kernel(e_ref, w_ref, b_ref, out_ref):
    out_ref[...] = lax.dot_general(
        e_ref[...], w_ref[...],
        (((1,), (1,)), ((), ())),
        preferred_element_type=jnp.float32,
    ) + b_ref[...]


def _project(embeds, W, b):
    return pl.pallas_call(
        _proj_kernel,
        grid=(N_BLOCKS,),
        in_specs=[
            pl.BlockSpec((BT, HIDDEN), lambda i: (i, 0)),
            pl.BlockSpec((VOCAB, HIDDEN), lambda i: (0, 0)),
            pl.BlockSpec((1, VOCAB), lambda i: (0, 0)),
        ],
        out_specs=pl.BlockSpec((BT, VOCAB), lambda i: (i, 0)),
        out_shape=jax.ShapeDtypeStruct((N_TOKENS, VOCAB), jnp.float32),
    )(embeds, W, b.reshape(1, VOCAB))


def kernel(input_ids, positions, emb_table, W, b):
    del positions
    ids = input_ids.reshape(-1).astype(jnp.int32)
    embeds = _sc_gather(ids, emb_table)
    out = _project(embeds, W, b)
    return out.reshape(input_ids.shape[0], input_ids.shape[1], VOCAB)
